# baseline (speedup 1.0000x reference)
import functools

import jax
import jax.numpy as jnp
from jax import lax
from jax.experimental import pallas as pl
from jax.experimental.pallas import tpu as pltpu
from jax.experimental.pallas import tpu_sc as plsc

NC = 2
NS = 16
NW = NC * NS
CHUNK = 128
GBG = 4
GBS = 2
GRP = NW * CHUNK * GBG
PADROWS = 112


def _sc_mesh():
    return plsc.VectorSubcoreMesh(core_axis_name="c", subcore_axis_name="s")


def _make_gather(N, EP, D):
    KMAX = EP // (NW * CHUNK * GBG)

    NCHK = EP // CHUNK

    @functools.partial(
        pl.kernel,
        out_type=jax.ShapeDtypeStruct((NCHK, CHUNK, D), jnp.float32),
        mesh=_sc_mesh(),
        scratch_types=[
            pltpu.VMEM((GBG, CHUNK), jnp.int32),
            pltpu.VMEM((GBG, CHUNK, D), jnp.float32),
            pltpu.SemaphoreType.DMA,
            pltpu.SemaphoreType.DMA,
            pltpu.SemaphoreType.DMA,
        ],
    )
    def gath(s_hbm, row2d_hbm, msg, rowbuf, gbuf, sem_i, sem_g, sem_w):
        c = lax.axis_index("c")
        sid = lax.axis_index("s")
        w = c * NS + sid

        @pl.loop(0, KMAX)
        def gbody(g):
            base = (g * NW + w) * GBG
            pltpu.async_copy(row2d_hbm.at[pl.ds(base, GBG)], rowbuf, sem_i)
            pltpu.make_async_copy(row2d_hbm.at[pl.ds(base, GBG)], rowbuf,
                                  sem_i).wait()
            for i in range(GBG):
                pltpu.async_copy(s_hbm.at[rowbuf.at[i]], gbuf.at[i], sem_g)
            for i in range(GBG):
                pltpu.make_async_copy(s_hbm.at[rowbuf.at[i]], gbuf.at[i],
                                      sem_g).wait()
            pltpu.async_copy(gbuf, msg.at[pl.ds(base, GBG)], sem_w)
            pltpu.make_async_copy(gbuf, msg.at[pl.ds(base, GBG)], sem_w).wait()

    return gath


def _make_scatter(N, EP, D):
    KMAX = EP // (NW * CHUNK * GBS)
    NA = N + PADROWS
    NZ = NA // CHUNK
    ZJ = (NZ + NS - 1) // NS
    OR = 80
    NO = N // OR
    OJ = (NO + NS - 1) // NS

    @functools.partial(
        pl.kernel,
        out_type=(jax.ShapeDtypeStruct((N, D), jnp.float32),
                  jax.ShapeDtypeStruct((N, D), jnp.float32)),
        mesh=_sc_mesh(),
        scratch_types=[
            pltpu.VMEM((GBS, CHUNK), jnp.int32),
            pltpu.VMEM((GBS, CHUNK, D), jnp.float32),
            pltpu.VMEM_SHARED((N + PADROWS, D), jnp.float32),
            pltpu.SemaphoreType.DMA,
            pltpu.SemaphoreType.DMA,
            pltpu.SemaphoreType.DMA,
        ],
    )
    def scat(msg_hbm, col2d_hbm, zeros_hbm, out_a, out_b,
             colbuf, gbuf, acc, sem_i, sem_m, sem_s):
        c = lax.axis_index("c")
        sid = lax.axis_index("s")
        w = c * NS + sid

        pltpu.sync_copy(zeros_hbm, gbuf.at[0])

        @pl.loop(0, ZJ)
        def zbody(j):
            ch = sid + NS * j

            @pl.when(ch < NZ)
            def _():
                pltpu.sync_copy(gbuf.at[0], acc.at[pl.ds(ch * CHUNK, CHUNK)])

        plsc.subcore_barrier()

        @pl.loop(0, KMAX)
        def gbody(g):
            base = (g * NW + w) * GBS
            pltpu.async_copy(col2d_hbm.at[pl.ds(base, GBS)], colbuf, sem_i)
            pltpu.async_copy(msg_hbm.at[pl.ds(base, GBS)], gbuf, sem_m)
            pltpu.make_async_copy(col2d_hbm.at[pl.ds(base, GBS)], colbuf,
                                  sem_i).wait()
            pltpu.make_async_copy(msg_hbm.at[pl.ds(base, GBS)], gbuf,
                                  sem_m).wait()
            for i in range(GBS):
                pltpu.async_copy(gbuf.at[i], acc.at[colbuf.at[i]], sem_s,
                                 add=True)
            for i in range(GBS):
                pltpu.make_async_copy(gbuf.at[i], acc.at[colbuf.at[i]],
                                      sem_s).wait()

        plsc.subcore_barrier()

        @pl.loop(0, OJ)
        def obody(j):
            ch = sid + NS * j

            @pl.when(ch < NO)
            def _():
                rows = pl.ds(ch * OR, OR)
                stage = gbuf.at[0].at[pl.ds(0, OR)]
                pltpu.sync_copy(acc.at[rows], stage)

                @pl.when(c == 0)
                def _():
                    pltpu.sync_copy(stage, out_a.at[rows])

                @pl.when(c == 1)
                def _():
                    pltpu.sync_copy(stage, out_b.at[rows])

    return scat


def _make_deg(N, EP, D):
    KMAX = EP // (NW * CHUNK * GBG)
    NA = N + PADROWS
    NZ = NA // CHUNK
    ZJ = (NZ + NS - 1) // NS
    OR = 80
    NO = N // OR
    OJ = (NO + NS - 1) // NS

    @functools.partial(
        pl.kernel,
        out_type=(jax.ShapeDtypeStruct((N, D), jnp.float32),
                  jax.ShapeDtypeStruct((N, D), jnp.float32)),
        mesh=_sc_mesh(),
        scratch_types=[
            pltpu.VMEM((GBG, CHUNK), jnp.int32),
            pltpu.VMEM((CHUNK, D), jnp.float32),
            pltpu.VMEM_SHARED((N + PADROWS, D), jnp.float32),
            pltpu.SemaphoreType.DMA,
            pltpu.SemaphoreType.DMA,
        ],
    )
    def deg(col2d_hbm, ones_hbm, zeros_hbm, out_a, out_b,
            colbuf, onesbuf, acc, sem_i, sem_s):
        c = lax.axis_index("c")
        sid = lax.axis_index("s")
        w = c * NS + sid

        pltpu.sync_copy(zeros_hbm, onesbuf)

        @pl.loop(0, ZJ)
        def zbody(j):
            ch = sid + NS * j

            @pl.when(ch < NZ)
            def _():
                pltpu.sync_copy(onesbuf, acc.at[pl.ds(ch * CHUNK, CHUNK)])

        plsc.subcore_barrier()
        pltpu.sync_copy(ones_hbm, onesbuf)

        @pl.loop(0, KMAX)
        def gbody(g):
            base = (g * NW + w) * GBG
            pltpu.async_copy(col2d_hbm.at[pl.ds(base, GBG)], colbuf, sem_i)
            pltpu.make_async_copy(col2d_hbm.at[pl.ds(base, GBG)], colbuf,
                                  sem_i).wait()
            for i in range(GBG):
                pltpu.async_copy(onesbuf, acc.at[colbuf.at[i]], sem_s,
                                 add=True)
            for i in range(GBG):
                pltpu.make_async_copy(onesbuf, acc.at[colbuf.at[i]],
                                      sem_s).wait()

        plsc.subcore_barrier()

        @pl.loop(0, OJ)
        def obody(j):
            ch = sid + NS * j

            @pl.when(ch < NO)
            def _():
                rows = pl.ds(ch * OR, OR)
                stage = onesbuf.at[pl.ds(0, OR)]
                pltpu.sync_copy(acc.at[rows], stage)

                @pl.when(c == 0)
                def _():
                    pltpu.sync_copy(stage, out_a.at[rows])

                @pl.when(c == 1)
                def _():
                    pltpu.sync_copy(stage, out_b.at[rows])

    return deg


def _dinv_from(da_ref, db_ref):
    return lax.rsqrt(da_ref[:, 0:1] + db_ref[:, 0:1] + 1.0)


def _mm_first(x, W, b, dega, degb, BLK=1000):
    N, D = x.shape

    def body(x_ref, w_ref, b_ref, da_ref, db_ref, o_ref):
        dinv = _dinv_from(da_ref, db_ref)
        z = jnp.dot(x_ref[:, :], w_ref[:, :],
                    preferred_element_type=jnp.float32) + b_ref[0:1, :]
        o_ref[:, :] = z * dinv

    return pl.pallas_call(
        body,
        grid=(N // BLK,),
        in_specs=[
            pl.BlockSpec((BLK, D), lambda i: (i, 0)),
            pl.BlockSpec((D, D), lambda i: (0, 0)),
            pl.BlockSpec((1, D), lambda i: (0, 0)),
            pl.BlockSpec((BLK, D), lambda i: (i, 0)),
            pl.BlockSpec((BLK, D), lambda i: (i, 0)),
        ],
        out_specs=pl.BlockSpec((BLK, D), lambda i: (i, 0)),
        out_shape=jax.ShapeDtypeStruct((N, D), jnp.float32),
    )(x, W, b, dega, degb)


def _mm_mid(pa, pb, s_prev, W, b, dega, degb, BLK=1000):
    N, D = s_prev.shape

    def body(pa_ref, pb_ref, s_ref, w_ref, b_ref, da_ref, db_ref, o_ref):
        dinv = _dinv_from(da_ref, db_ref)
        h = jnp.maximum((pa_ref[:, :] + pb_ref[:, :] + s_ref[:, :]) * dinv, 0.0)
        z = jnp.dot(h, w_ref[:, :],
                    preferred_element_type=jnp.float32) + b_ref[0:1, :]
        o_ref[:, :] = z * dinv

    return pl.pallas_call(
        body,
        grid=(N // BLK,),
        in_specs=[
            pl.BlockSpec((BLK, D), lambda i: (i, 0)),
            pl.BlockSpec((BLK, D), lambda i: (i, 0)),
            pl.BlockSpec((BLK, D), lambda i: (i, 0)),
            pl.BlockSpec((D, D), lambda i: (0, 0)),
            pl.BlockSpec((1, D), lambda i: (0, 0)),
            pl.BlockSpec((BLK, D), lambda i: (i, 0)),
            pl.BlockSpec((BLK, D), lambda i: (i, 0)),
        ],
        out_specs=pl.BlockSpec((BLK, D), lambda i: (i, 0)),
        out_shape=jax.ShapeDtypeStruct((N, D), jnp.float32),
    )(pa, pb, s_prev, W, b, dega, degb)


def _final(pa, pb, s_prev, dega, degb, batch3d, Wh1, bh1, Wh2p, bh2p, BLK=1000):
    N, D = s_prev.shape
    GP = 128
    nblk = N // BLK

    def body(pa_ref, pb_ref, s_ref, da_ref, db_ref, bt_ref,
             wh1_ref, bh1_ref, wh2_ref, bh2_ref, o_ref, pool_acc, cnt_acc):
        i = pl.program_id(0)

        @pl.when(i == 0)
        def _():
            pool_acc[:, :] = jnp.zeros((GP, D), jnp.float32)
            cnt_acc[:, :] = jnp.zeros((GP, D), jnp.float32)

        dinv = _dinv_from(da_ref, db_ref)
        h = jnp.maximum((pa_ref[:, :] + pb_ref[:, :] + s_ref[:, :]) * dinv, 0.0)
        bt = jnp.broadcast_to(bt_ref[0], (GP, BLK))
        gid = lax.broadcasted_iota(jnp.int32, (GP, BLK), 0)
        onehot_t = jnp.where(bt == gid, 1.0, 0.0)
        pool_acc[:, :] += lax.dot_general(
            onehot_t, h, (((1,), (0,)), ((), ())),
            preferred_element_type=jnp.float32)
        cnt_acc[:, :] += lax.dot_general(
            onehot_t, jnp.ones((BLK, D), jnp.float32), (((1,), (0,)), ((), ())),
            preferred_element_type=jnp.float32)

        @pl.when(i == nblk - 1)
        def _():
            g = pool_acc[:, :] / jnp.maximum(cnt_acc[:, :], 1.0)
            g1 = jnp.maximum(
                jnp.dot(g, wh1_ref[:, :],
                        preferred_element_type=jnp.float32) + bh1_ref[0:1, :],
                0.0)
            o_ref[:, :] = jnp.dot(g1, wh2_ref[:, :],
                                  preferred_element_type=jnp.float32) + bh2_ref[0:1, :]

    return pl.pallas_call(
        body,
        grid=(nblk,),
        in_specs=[
            pl.BlockSpec((BLK, D), lambda i: (i, 0)),
            pl.BlockSpec((BLK, D), lambda i: (i, 0)),
            pl.BlockSpec((BLK, D), lambda i: (i, 0)),
            pl.BlockSpec((BLK, D), lambda i: (i, 0)),
            pl.BlockSpec((BLK, D), lambda i: (i, 0)),
            pl.BlockSpec((1, 1, BLK), lambda i: (i, 0, 0)),
            pl.BlockSpec((D, D), lambda i: (0, 0)),
            pl.BlockSpec((1, D), lambda i: (0, 0)),
            pl.BlockSpec((D, GP), lambda i: (0, 0)),
            pl.BlockSpec((1, GP), lambda i: (0, 0)),
        ],
        out_specs=pl.BlockSpec((GP, D), lambda i: (0, 0)),
        out_shape=jax.ShapeDtypeStruct((GP, D), jnp.float32),
        scratch_shapes=[
            pltpu.VMEM((GP, D), jnp.float32),
            pltpu.VMEM((GP, D), jnp.float32),
        ],
    )(pa, pb, s_prev, dega, degb, batch3d, Wh1, bh1, Wh2p, bh2p)


def kernel(x, edge_index, batch, W0, b0, W1, b1, W2, b2, Wh1, bh1, Wh2, bh2):
    N, D = x.shape
    E = edge_index.shape[1]
    G = 64
    EP = ((E + GRP - 1) // GRP) * GRP
    pad = EP - E
    row = jnp.concatenate(
        [edge_index[0], jnp.zeros((pad,), jnp.int32)]).reshape(EP // CHUNK, CHUNK)
    col = jnp.concatenate(
        [edge_index[1], jnp.full((pad,), N, jnp.int32)]).reshape(EP // CHUNK, CHUNK)

    zeros_d = jnp.zeros((CHUNK, D), jnp.float32)
    ones_d = jnp.ones((CHUNK, D), jnp.float32)
    batch3d = batch.reshape(N // 1000, 1, 1000)
    b0r = b0.reshape(1, D)
    b1r = b1.reshape(1, D)
    b2r = b2.reshape(1, D)
    bh1r = bh1.reshape(1, D)
    Wh2p = jnp.pad(Wh2, ((0, 0), (0, 128 - Wh2.shape[1])))
    bh2p = jnp.broadcast_to(bh2.reshape(1, 1), (1, 128))

    deg = _make_deg(N, EP, D)
    gath = _make_gather(N, EP, D)
    scat = _make_scatter(N, EP, D)

    dega, degb = deg(col, ones_d, zeros_d)

    s0 = _mm_first(x, W0, b0r, dega, degb)
    p0a, p0b = scat(gath(s0, row), col, zeros_d)
    s1 = _mm_mid(p0a, p0b, s0, W1, b1r, dega, degb)
    p1a, p1b = scat(gath(s1, row), col, zeros_d)
    s2 = _mm_mid(p1a, p1b, s1, W2, b2r, dega, degb)
    p2a, p2b = scat(gath(s2, row), col, zeros_d)

    out = _final(p2a, p2b, s2, dega, degb, batch3d, Wh1, bh1r, Wh2p, bh2p)
    return out[:G, 0]


# speedup vs baseline: 1.0702x; 1.0702x over previous
import functools

import jax
import jax.numpy as jnp
from jax import lax
from jax.experimental import pallas as pl
from jax.experimental.pallas import tpu as pltpu
from jax.experimental.pallas import tpu_sc as plsc

NC = 2
NS = 16
NW = NC * NS
CHUNK = 128
GBG = 4
GBS = 2
GRP = NW * CHUNK * GBG
PADROWS = 112


def _sc_mesh():
    return plsc.VectorSubcoreMesh(core_axis_name="c", subcore_axis_name="s")


def _make_gather(N, EP, D):
    KMAX = EP // (NW * CHUNK * GBG)

    NCHK = EP // CHUNK

    @functools.partial(
        pl.kernel,
        out_type=jax.ShapeDtypeStruct((NCHK, CHUNK, D), jnp.float32),
        mesh=_sc_mesh(),
        scratch_types=[
            pltpu.VMEM((GBG, CHUNK), jnp.int32),
            pltpu.VMEM((GBG, CHUNK, D), jnp.float32),
            pltpu.SemaphoreType.DMA,
            pltpu.SemaphoreType.DMA,
            pltpu.SemaphoreType.DMA,
        ],
    )
    def gath(s_hbm, row2d_hbm, msg, rowbuf, gbuf, sem_i, sem_g, sem_w):
        c = lax.axis_index("c")
        sid = lax.axis_index("s")
        w = c * NS + sid

        @pl.loop(0, KMAX)
        def gbody(g):
            base = (g * NW + w) * GBG
            pltpu.async_copy(row2d_hbm.at[pl.ds(base, GBG)], rowbuf, sem_i)
            pltpu.make_async_copy(row2d_hbm.at[pl.ds(base, GBG)], rowbuf,
                                  sem_i).wait()
            for i in range(GBG):
                pltpu.async_copy(s_hbm.at[rowbuf.at[i]], gbuf.at[i], sem_g)
            for i in range(GBG):
                pltpu.make_async_copy(s_hbm.at[rowbuf.at[i]], gbuf.at[i],
                                      sem_g).wait()
            pltpu.async_copy(gbuf, msg.at[pl.ds(base, GBG)], sem_w)
            pltpu.make_async_copy(gbuf, msg.at[pl.ds(base, GBG)], sem_w).wait()

    return gath


def _make_scatter(N, EP, D):
    KMAX = EP // (NW * CHUNK)
    NA = N + PADROWS
    NZ = NA // CHUNK
    ZJ = (NZ + NS - 1) // NS
    OR = 80
    NO = N // OR
    OJ = (NO + NS - 1) // NS

    @functools.partial(
        pl.kernel,
        out_type=(jax.ShapeDtypeStruct((N, D), jnp.float32),
                  jax.ShapeDtypeStruct((N, D), jnp.float32)),
        mesh=_sc_mesh(),
        scratch_types=[
            pltpu.VMEM((2, CHUNK), jnp.int32),
            pltpu.VMEM((2, CHUNK, D), jnp.float32),
            pltpu.VMEM_SHARED((N + PADROWS, D), jnp.float32),
            pltpu.SemaphoreType.DMA,
            pltpu.SemaphoreType.DMA,
            pltpu.SemaphoreType.DMA,
        ],
    )
    def scat(msg_hbm, col2d_hbm, zeros_hbm, out_a, out_b,
             colbuf, gbuf, acc, sem_i, sem_m, sem_s):
        c = lax.axis_index("c")
        sid = lax.axis_index("s")
        w = c * NS + sid

        pltpu.sync_copy(zeros_hbm, gbuf.at[0])

        @pl.loop(0, ZJ)
        def zbody(j):
            ch = sid + NS * j

            @pl.when(ch < NZ)
            def _():
                pltpu.sync_copy(gbuf.at[0], acc.at[pl.ds(ch * CHUNK, CHUNK)])

        plsc.subcore_barrier()

        def chunk_of(g):
            return g * NW + w

        pltpu.async_copy(col2d_hbm.at[pl.ds(chunk_of(0), 1)],
                         colbuf.at[pl.ds(0, 1)], sem_i)
        pltpu.async_copy(msg_hbm.at[pl.ds(chunk_of(0), 1)],
                         gbuf.at[pl.ds(0, 1)], sem_m)

        @pl.loop(0, KMAX)
        def gbody(g):
            cur = lax.rem(g, 2)
            nxt = lax.rem(g + 1, 2)

            @pl.when(g + 1 < KMAX)
            def _():
                pltpu.async_copy(col2d_hbm.at[pl.ds(chunk_of(g + 1), 1)],
                                 colbuf.at[pl.ds(nxt, 1)], sem_i)
                pltpu.async_copy(msg_hbm.at[pl.ds(chunk_of(g + 1), 1)],
                                 gbuf.at[pl.ds(nxt, 1)], sem_m)

            pltpu.make_async_copy(col2d_hbm.at[pl.ds(chunk_of(g), 1)],
                                  colbuf.at[pl.ds(cur, 1)], sem_i).wait()
            pltpu.make_async_copy(msg_hbm.at[pl.ds(chunk_of(g), 1)],
                                  gbuf.at[pl.ds(cur, 1)], sem_m).wait()
            pltpu.async_copy(gbuf.at[cur], acc.at[colbuf.at[cur]], sem_s,
                             add=True)
            pltpu.make_async_copy(gbuf.at[cur], acc.at[colbuf.at[cur]],
                                  sem_s).wait()

        plsc.subcore_barrier()

        @pl.loop(0, OJ)
        def obody(j):
            ch = sid + NS * j

            @pl.when(ch < NO)
            def _():
                rows = pl.ds(ch * OR, OR)
                stage = gbuf.at[0].at[pl.ds(0, OR)]
                pltpu.sync_copy(acc.at[rows], stage)

                @pl.when(c == 0)
                def _():
                    pltpu.sync_copy(stage, out_a.at[rows])

                @pl.when(c == 1)
                def _():
                    pltpu.sync_copy(stage, out_b.at[rows])

    return scat


def _make_deg(N, EP, D):
    KMAX = EP // (NW * CHUNK * GBG)
    NA = N + PADROWS
    NZ = NA // CHUNK
    ZJ = (NZ + NS - 1) // NS
    OR = 80
    NO = N // OR
    OJ = (NO + NS - 1) // NS

    @functools.partial(
        pl.kernel,
        out_type=(jax.ShapeDtypeStruct((N, D), jnp.float32),
                  jax.ShapeDtypeStruct((N, D), jnp.float32)),
        mesh=_sc_mesh(),
        scratch_types=[
            pltpu.VMEM((GBG, CHUNK), jnp.int32),
            pltpu.VMEM((CHUNK, D), jnp.float32),
            pltpu.VMEM_SHARED((N + PADROWS, D), jnp.float32),
            pltpu.SemaphoreType.DMA,
            pltpu.SemaphoreType.DMA,
        ],
    )
    def deg(col2d_hbm, ones_hbm, zeros_hbm, out_a, out_b,
            colbuf, onesbuf, acc, sem_i, sem_s):
        c = lax.axis_index("c")
        sid = lax.axis_index("s")
        w = c * NS + sid

        pltpu.sync_copy(zeros_hbm, onesbuf)

        @pl.loop(0, ZJ)
        def zbody(j):
            ch = sid + NS * j

            @pl.when(ch < NZ)
            def _():
                pltpu.sync_copy(onesbuf, acc.at[pl.ds(ch * CHUNK, CHUNK)])

        plsc.subcore_barrier()
        pltpu.sync_copy(ones_hbm, onesbuf)

        @pl.loop(0, KMAX)
        def gbody(g):
            base = (g * NW + w) * GBG
            pltpu.async_copy(col2d_hbm.at[pl.ds(base, GBG)], colbuf, sem_i)
            pltpu.make_async_copy(col2d_hbm.at[pl.ds(base, GBG)], colbuf,
                                  sem_i).wait()
            for i in range(GBG):
                pltpu.async_copy(onesbuf, acc.at[colbuf.at[i]], sem_s,
                                 add=True)
            for i in range(GBG):
                pltpu.make_async_copy(onesbuf, acc.at[colbuf.at[i]],
                                      sem_s).wait()

        plsc.subcore_barrier()

        @pl.loop(0, OJ)
        def obody(j):
            ch = sid + NS * j

            @pl.when(ch < NO)
            def _():
                rows = pl.ds(ch * OR, OR)
                stage = onesbuf.at[pl.ds(0, OR)]
                pltpu.sync_copy(acc.at[rows], stage)

                @pl.when(c == 0)
                def _():
                    pltpu.sync_copy(stage, out_a.at[rows])

                @pl.when(c == 1)
                def _():
                    pltpu.sync_copy(stage, out_b.at[rows])

    return deg


def _dinv_from(da_ref, db_ref):
    return lax.rsqrt(da_ref[:, 0:1] + db_ref[:, 0:1] + 1.0)


def _mm_first(x, W, b, dega, degb, BLK=1000):
    N, D = x.shape

    def body(x_ref, w_ref, b_ref, da_ref, db_ref, o_ref):
        dinv = _dinv_from(da_ref, db_ref)
        z = jnp.dot(x_ref[:, :], w_ref[:, :],
                    preferred_element_type=jnp.float32) + b_ref[0:1, :]
        o_ref[:, :] = z * dinv

    return pl.pallas_call(
        body,
        grid=(N // BLK,),
        in_specs=[
            pl.BlockSpec((BLK, D), lambda i: (i, 0)),
            pl.BlockSpec((D, D), lambda i: (0, 0)),
            pl.BlockSpec((1, D), lambda i: (0, 0)),
            pl.BlockSpec((BLK, D), lambda i: (i, 0)),
            pl.BlockSpec((BLK, D), lambda i: (i, 0)),
        ],
        out_specs=pl.BlockSpec((BLK, D), lambda i: (i, 0)),
        out_shape=jax.ShapeDtypeStruct((N, D), jnp.float32),
    )(x, W, b, dega, degb)


def _mm_mid(pa, pb, s_prev, W, b, dega, degb, BLK=1000):
    N, D = s_prev.shape

    def body(pa_ref, pb_ref, s_ref, w_ref, b_ref, da_ref, db_ref, o_ref):
        dinv = _dinv_from(da_ref, db_ref)
        h = jnp.maximum((pa_ref[:, :] + pb_ref[:, :] + s_ref[:, :]) * dinv, 0.0)
        z = jnp.dot(h, w_ref[:, :],
                    preferred_element_type=jnp.float32) + b_ref[0:1, :]
        o_ref[:, :] = z * dinv

    return pl.pallas_call(
        body,
        grid=(N // BLK,),
        in_specs=[
            pl.BlockSpec((BLK, D), lambda i: (i, 0)),
            pl.BlockSpec((BLK, D), lambda i: (i, 0)),
            pl.BlockSpec((BLK, D), lambda i: (i, 0)),
            pl.BlockSpec((D, D), lambda i: (0, 0)),
            pl.BlockSpec((1, D), lambda i: (0, 0)),
            pl.BlockSpec((BLK, D), lambda i: (i, 0)),
            pl.BlockSpec((BLK, D), lambda i: (i, 0)),
        ],
        out_specs=pl.BlockSpec((BLK, D), lambda i: (i, 0)),
        out_shape=jax.ShapeDtypeStruct((N, D), jnp.float32),
    )(pa, pb, s_prev, W, b, dega, degb)


def _final(pa, pb, s_prev, dega, degb, batch3d, Wh1, bh1, Wh2p, bh2p, BLK=1000):
    N, D = s_prev.shape
    GP = 128
    nblk = N // BLK

    def body(pa_ref, pb_ref, s_ref, da_ref, db_ref, bt_ref,
             wh1_ref, bh1_ref, wh2_ref, bh2_ref, o_ref, pool_acc, cnt_acc):
        i = pl.program_id(0)

        @pl.when(i == 0)
        def _():
            pool_acc[:, :] = jnp.zeros((GP, D), jnp.float32)
            cnt_acc[:, :] = jnp.zeros((GP, D), jnp.float32)

        dinv = _dinv_from(da_ref, db_ref)
        h = jnp.maximum((pa_ref[:, :] + pb_ref[:, :] + s_ref[:, :]) * dinv, 0.0)
        bt = jnp.broadcast_to(bt_ref[0], (GP, BLK))
        gid = lax.broadcasted_iota(jnp.int32, (GP, BLK), 0)
        onehot_t = jnp.where(bt == gid, 1.0, 0.0)
        pool_acc[:, :] += lax.dot_general(
            onehot_t, h, (((1,), (0,)), ((), ())),
            preferred_element_type=jnp.float32)
        cnt_acc[:, :] += lax.dot_general(
            onehot_t, jnp.ones((BLK, D), jnp.float32), (((1,), (0,)), ((), ())),
            preferred_element_type=jnp.float32)

        @pl.when(i == nblk - 1)
        def _():
            g = pool_acc[:, :] / jnp.maximum(cnt_acc[:, :], 1.0)
            g1 = jnp.maximum(
                jnp.dot(g, wh1_ref[:, :],
                        preferred_element_type=jnp.float32) + bh1_ref[0:1, :],
                0.0)
            o_ref[:, :] = jnp.dot(g1, wh2_ref[:, :],
                                  preferred_element_type=jnp.float32) + bh2_ref[0:1, :]

    return pl.pallas_call(
        body,
        grid=(nblk,),
        in_specs=[
            pl.BlockSpec((BLK, D), lambda i: (i, 0)),
            pl.BlockSpec((BLK, D), lambda i: (i, 0)),
            pl.BlockSpec((BLK, D), lambda i: (i, 0)),
            pl.BlockSpec((BLK, D), lambda i: (i, 0)),
            pl.BlockSpec((BLK, D), lambda i: (i, 0)),
            pl.BlockSpec((1, 1, BLK), lambda i: (i, 0, 0)),
            pl.BlockSpec((D, D), lambda i: (0, 0)),
            pl.BlockSpec((1, D), lambda i: (0, 0)),
            pl.BlockSpec((D, GP), lambda i: (0, 0)),
            pl.BlockSpec((1, GP), lambda i: (0, 0)),
        ],
        out_specs=pl.BlockSpec((GP, D), lambda i: (0, 0)),
        out_shape=jax.ShapeDtypeStruct((GP, D), jnp.float32),
        scratch_shapes=[
            pltpu.VMEM((GP, D), jnp.float32),
            pltpu.VMEM((GP, D), jnp.float32),
        ],
    )(pa, pb, s_prev, dega, degb, batch3d, Wh1, bh1, Wh2p, bh2p)


def kernel(x, edge_index, batch, W0, b0, W1, b1, W2, b2, Wh1, bh1, Wh2, bh2):
    N, D = x.shape
    E = edge_index.shape[1]
    G = 64
    EP = ((E + GRP - 1) // GRP) * GRP
    pad = EP - E
    row = jnp.concatenate(
        [edge_index[0], jnp.zeros((pad,), jnp.int32)]).reshape(EP // CHUNK, CHUNK)
    col = jnp.concatenate(
        [edge_index[1], jnp.full((pad,), N, jnp.int32)]).reshape(EP // CHUNK, CHUNK)

    zeros_d = jnp.zeros((CHUNK, D), jnp.float32)
    ones_d = jnp.ones((CHUNK, D), jnp.float32)
    batch3d = batch.reshape(N // 1000, 1, 1000)
    b0r = b0.reshape(1, D)
    b1r = b1.reshape(1, D)
    b2r = b2.reshape(1, D)
    bh1r = bh1.reshape(1, D)
    Wh2p = jnp.pad(Wh2, ((0, 0), (0, 128 - Wh2.shape[1])))
    bh2p = jnp.broadcast_to(bh2.reshape(1, 1), (1, 128))

    deg = _make_deg(N, EP, D)
    gath = _make_gather(N, EP, D)
    scat = _make_scatter(N, EP, D)

    dega, degb = deg(col, ones_d, zeros_d)

    s0 = _mm_first(x, W0, b0r, dega, degb)
    p0a, p0b = scat(gath(s0, row), col, zeros_d)
    s1 = _mm_mid(p0a, p0b, s0, W1, b1r, dega, degb)
    p1a, p1b = scat(gath(s1, row), col, zeros_d)
    s2 = _mm_mid(p1a, p1b, s1, W2, b2r, dega, degb)
    p2a, p2b = scat(gath(s2, row), col, zeros_d)

    out = _final(p2a, p2b, s2, dega, degb, batch3d, Wh1, bh1r, Wh2p, bh2p)
    return out[:G, 0]


# speedup vs baseline: 2.6552x; 2.4810x over previous
import functools

import jax
import jax.numpy as jnp
from jax import lax
from jax.experimental import pallas as pl
from jax.experimental.pallas import tpu as pltpu
from jax.experimental.pallas import tpu_sc as plsc

NC = 2
NS = 16
NW = NC * NS
CHUNK = 128
GBG = 4
GBS = 2
GRP = NW * CHUNK * GBG
PADROWS = 112


def _sc_mesh():
    return plsc.VectorSubcoreMesh(core_axis_name="c", subcore_axis_name="s")


def _make_gather(N, EP, D):
    KMAX = EP // (NW * CHUNK)
    NCHK = EP // CHUNK
    SR = 80
    NSC = N // SR

    @functools.partial(
        pl.kernel,
        out_type=jax.ShapeDtypeStruct((NCHK, CHUNK, D), jnp.float32),
        mesh=_sc_mesh(),
        scratch_types=[
            pltpu.VMEM((2, CHUNK), jnp.int32),
            pltpu.VMEM((2, CHUNK, D), jnp.float32),
            pltpu.VMEM_SHARED((N, D), jnp.float32),
            pltpu.SemaphoreType.DMA,
            pltpu.SemaphoreType.DMA,
            pltpu.SemaphoreType.DMA,
        ],
    )
    def gath(s_hbm, row2d_hbm, msg, rowbuf, gbuf, s_sp, sem_i, sem_g, sem_w):
        c = lax.axis_index("c")
        sid = lax.axis_index("s")
        w = c * NS + sid

        @pl.loop(0, (NSC + NS - 1) // NS)
        def lbody(j):
            ch = sid + NS * j

            @pl.when(ch < NSC)
            def _():
                pltpu.sync_copy(s_hbm.at[pl.ds(ch * SR, SR)],
                                s_sp.at[pl.ds(ch * SR, SR)])

        plsc.subcore_barrier()

        def chunk_of(g):
            return g * NW + w

        pltpu.async_copy(row2d_hbm.at[pl.ds(chunk_of(0), 1)],
                         rowbuf.at[pl.ds(0, 1)], sem_i)

        @pl.loop(0, KMAX)
        def gbody(g):
            cur = lax.rem(g, 2)
            nxt = lax.rem(g + 1, 2)

            @pl.when(g + 1 < KMAX)
            def _():
                pltpu.async_copy(row2d_hbm.at[pl.ds(chunk_of(g + 1), 1)],
                                 rowbuf.at[pl.ds(nxt, 1)], sem_i)

            pltpu.make_async_copy(row2d_hbm.at[pl.ds(chunk_of(g), 1)],
                                  rowbuf.at[pl.ds(cur, 1)], sem_i).wait()

            @pl.when(g >= 2)
            def _():
                pltpu.make_async_copy(gbuf.at[pl.ds(cur, 1)],
                                      msg.at[pl.ds(chunk_of(g - 2), 1)],
                                      sem_w).wait()

            pltpu.async_copy(s_sp.at[rowbuf.at[cur]], gbuf.at[cur],
                             sem_g).wait()
            pltpu.async_copy(gbuf.at[pl.ds(cur, 1)],
                             msg.at[pl.ds(chunk_of(g), 1)], sem_w)

        pltpu.make_async_copy(gbuf.at[pl.ds(0, 1)],
                              msg.at[pl.ds(chunk_of(KMAX - 2), 1)],
                              sem_w).wait()
        pltpu.make_async_copy(gbuf.at[pl.ds(1, 1)],
                              msg.at[pl.ds(chunk_of(KMAX - 1), 1)],
                              sem_w).wait()

    return gath


def _make_scatter(N, EP, D):
    KMAX = EP // (NW * CHUNK)
    NA = N + PADROWS
    NZ = NA // CHUNK
    ZJ = (NZ + NS - 1) // NS
    OR = 80
    NO = N // OR
    OJ = (NO + NS - 1) // NS

    @functools.partial(
        pl.kernel,
        out_type=(jax.ShapeDtypeStruct((N, D), jnp.float32),
                  jax.ShapeDtypeStruct((N, D), jnp.float32)),
        mesh=_sc_mesh(),
        scratch_types=[
            pltpu.VMEM((2, CHUNK), jnp.int32),
            pltpu.VMEM((2, CHUNK, D), jnp.float32),
            pltpu.VMEM_SHARED((N + PADROWS, D), jnp.float32),
            pltpu.SemaphoreType.DMA,
            pltpu.SemaphoreType.DMA,
            pltpu.SemaphoreType.DMA,
        ],
    )
    def scat(msg_hbm, col2d_hbm, zeros_hbm, out_a, out_b,
             colbuf, gbuf, acc, sem_i, sem_m, sem_s):
        c = lax.axis_index("c")
        sid = lax.axis_index("s")
        w = c * NS + sid

        pltpu.sync_copy(zeros_hbm, gbuf.at[0])

        @pl.loop(0, ZJ)
        def zbody(j):
            ch = sid + NS * j

            @pl.when(ch < NZ)
            def _():
                pltpu.sync_copy(gbuf.at[0], acc.at[pl.ds(ch * CHUNK, CHUNK)])

        plsc.subcore_barrier()

        def chunk_of(g):
            return g * NW + w

        pltpu.async_copy(col2d_hbm.at[pl.ds(chunk_of(0), 1)],
                         colbuf.at[pl.ds(0, 1)], sem_i)
        pltpu.async_copy(msg_hbm.at[pl.ds(chunk_of(0), 1)],
                         gbuf.at[pl.ds(0, 1)], sem_m)

        @pl.loop(0, KMAX)
        def gbody(g):
            cur = lax.rem(g, 2)
            nxt = lax.rem(g + 1, 2)

            @pl.when(g + 1 < KMAX)
            def _():
                pltpu.async_copy(col2d_hbm.at[pl.ds(chunk_of(g + 1), 1)],
                                 colbuf.at[pl.ds(nxt, 1)], sem_i)
                pltpu.async_copy(msg_hbm.at[pl.ds(chunk_of(g + 1), 1)],
                                 gbuf.at[pl.ds(nxt, 1)], sem_m)

            pltpu.make_async_copy(col2d_hbm.at[pl.ds(chunk_of(g), 1)],
                                  colbuf.at[pl.ds(cur, 1)], sem_i).wait()
            pltpu.make_async_copy(msg_hbm.at[pl.ds(chunk_of(g), 1)],
                                  gbuf.at[pl.ds(cur, 1)], sem_m).wait()
            pltpu.async_copy(gbuf.at[cur], acc.at[colbuf.at[cur]], sem_s,
                             add=True)
            pltpu.make_async_copy(gbuf.at[cur], acc.at[colbuf.at[cur]],
                                  sem_s).wait()

        plsc.subcore_barrier()

        @pl.loop(0, OJ)
        def obody(j):
            ch = sid + NS * j

            @pl.when(ch < NO)
            def _():
                rows = pl.ds(ch * OR, OR)
                stage = gbuf.at[0].at[pl.ds(0, OR)]
                pltpu.sync_copy(acc.at[rows], stage)

                @pl.when(c == 0)
                def _():
                    pltpu.sync_copy(stage, out_a.at[rows])

                @pl.when(c == 1)
                def _():
                    pltpu.sync_copy(stage, out_b.at[rows])

    return scat


def _make_deg(N, EP, D):
    KMAX = EP // (NW * CHUNK * GBG)
    NA = N + PADROWS
    NZ = NA // CHUNK
    ZJ = (NZ + NS - 1) // NS
    OR = 80
    NO = N // OR
    OJ = (NO + NS - 1) // NS

    @functools.partial(
        pl.kernel,
        out_type=(jax.ShapeDtypeStruct((N, D), jnp.float32),
                  jax.ShapeDtypeStruct((N, D), jnp.float32)),
        mesh=_sc_mesh(),
        scratch_types=[
            pltpu.VMEM((GBG, CHUNK), jnp.int32),
            pltpu.VMEM((CHUNK, D), jnp.float32),
            pltpu.VMEM_SHARED((N + PADROWS, D), jnp.float32),
            pltpu.SemaphoreType.DMA,
            pltpu.SemaphoreType.DMA,
        ],
    )
    def deg(col2d_hbm, ones_hbm, zeros_hbm, out_a, out_b,
            colbuf, onesbuf, acc, sem_i, sem_s):
        c = lax.axis_index("c")
        sid = lax.axis_index("s")
        w = c * NS + sid

        pltpu.sync_copy(zeros_hbm, onesbuf)

        @pl.loop(0, ZJ)
        def zbody(j):
            ch = sid + NS * j

            @pl.when(ch < NZ)
            def _():
                pltpu.sync_copy(onesbuf, acc.at[pl.ds(ch * CHUNK, CHUNK)])

        plsc.subcore_barrier()
        pltpu.sync_copy(ones_hbm, onesbuf)

        @pl.loop(0, KMAX)
        def gbody(g):
            base = (g * NW + w) * GBG
            pltpu.async_copy(col2d_hbm.at[pl.ds(base, GBG)], colbuf, sem_i)
            pltpu.make_async_copy(col2d_hbm.at[pl.ds(base, GBG)], colbuf,
                                  sem_i).wait()
            for i in range(GBG):
                pltpu.async_copy(onesbuf, acc.at[colbuf.at[i]], sem_s,
                                 add=True)
            for i in range(GBG):
                pltpu.make_async_copy(onesbuf, acc.at[colbuf.at[i]],
                                      sem_s).wait()

        plsc.subcore_barrier()

        @pl.loop(0, OJ)
        def obody(j):
            ch = sid + NS * j

            @pl.when(ch < NO)
            def _():
                rows = pl.ds(ch * OR, OR)
                stage = onesbuf.at[pl.ds(0, OR)]
                pltpu.sync_copy(acc.at[rows], stage)

                @pl.when(c == 0)
                def _():
                    pltpu.sync_copy(stage, out_a.at[rows])

                @pl.when(c == 1)
                def _():
                    pltpu.sync_copy(stage, out_b.at[rows])

    return deg


def _dinv_from(da_ref, db_ref):
    return lax.rsqrt(da_ref[:, 0:1] + db_ref[:, 0:1] + 1.0)


def _mm_first(x, W, b, dega, degb, BLK=1000):
    N, D = x.shape

    def body(x_ref, w_ref, b_ref, da_ref, db_ref, o_ref):
        dinv = _dinv_from(da_ref, db_ref)
        z = jnp.dot(x_ref[:, :], w_ref[:, :],
                    preferred_element_type=jnp.float32) + b_ref[0:1, :]
        o_ref[:, :] = z * dinv

    return pl.pallas_call(
        body,
        grid=(N // BLK,),
        in_specs=[
            pl.BlockSpec((BLK, D), lambda i: (i, 0)),
            pl.BlockSpec((D, D), lambda i: (0, 0)),
            pl.BlockSpec((1, D), lambda i: (0, 0)),
            pl.BlockSpec((BLK, D), lambda i: (i, 0)),
            pl.BlockSpec((BLK, D), lambda i: (i, 0)),
        ],
        out_specs=pl.BlockSpec((BLK, D), lambda i: (i, 0)),
        out_shape=jax.ShapeDtypeStruct((N, D), jnp.float32),
    )(x, W, b, dega, degb)


def _mm_mid(pa, pb, s_prev, W, b, dega, degb, BLK=1000):
    N, D = s_prev.shape

    def body(pa_ref, pb_ref, s_ref, w_ref, b_ref, da_ref, db_ref, o_ref):
        dinv = _dinv_from(da_ref, db_ref)
        h = jnp.maximum((pa_ref[:, :] + pb_ref[:, :] + s_ref[:, :]) * dinv, 0.0)
        z = jnp.dot(h, w_ref[:, :],
                    preferred_element_type=jnp.float32) + b_ref[0:1, :]
        o_ref[:, :] = z * dinv

    return pl.pallas_call(
        body,
        grid=(N // BLK,),
        in_specs=[
            pl.BlockSpec((BLK, D), lambda i: (i, 0)),
            pl.BlockSpec((BLK, D), lambda i: (i, 0)),
            pl.BlockSpec((BLK, D), lambda i: (i, 0)),
            pl.BlockSpec((D, D), lambda i: (0, 0)),
            pl.BlockSpec((1, D), lambda i: (0, 0)),
            pl.BlockSpec((BLK, D), lambda i: (i, 0)),
            pl.BlockSpec((BLK, D), lambda i: (i, 0)),
        ],
        out_specs=pl.BlockSpec((BLK, D), lambda i: (i, 0)),
        out_shape=jax.ShapeDtypeStruct((N, D), jnp.float32),
    )(pa, pb, s_prev, W, b, dega, degb)


def _final(pa, pb, s_prev, dega, degb, batch3d, Wh1, bh1, Wh2p, bh2p, BLK=1000):
    N, D = s_prev.shape
    GP = 128
    nblk = N // BLK

    def body(pa_ref, pb_ref, s_ref, da_ref, db_ref, bt_ref,
             wh1_ref, bh1_ref, wh2_ref, bh2_ref, o_ref, pool_acc, cnt_acc):
        i = pl.program_id(0)

        @pl.when(i == 0)
        def _():
            pool_acc[:, :] = jnp.zeros((GP, D), jnp.float32)
            cnt_acc[:, :] = jnp.zeros((GP, D), jnp.float32)

        dinv = _dinv_from(da_ref, db_ref)
        h = jnp.maximum((pa_ref[:, :] + pb_ref[:, :] + s_ref[:, :]) * dinv, 0.0)
        bt = jnp.broadcast_to(bt_ref[0], (GP, BLK))
        gid = lax.broadcasted_iota(jnp.int32, (GP, BLK), 0)
        onehot_t = jnp.where(bt == gid, 1.0, 0.0)
        pool_acc[:, :] += lax.dot_general(
            onehot_t, h, (((1,), (0,)), ((), ())),
            preferred_element_type=jnp.float32)
        cnt_acc[:, :] += lax.dot_general(
            onehot_t, jnp.ones((BLK, D), jnp.float32), (((1,), (0,)), ((), ())),
            preferred_element_type=jnp.float32)

        @pl.when(i == nblk - 1)
        def _():
            g = pool_acc[:, :] / jnp.maximum(cnt_acc[:, :], 1.0)
            g1 = jnp.maximum(
                jnp.dot(g, wh1_ref[:, :],
                        preferred_element_type=jnp.float32) + bh1_ref[0:1, :],
                0.0)
            o_ref[:, :] = jnp.dot(g1, wh2_ref[:, :],
                                  preferred_element_type=jnp.float32) + bh2_ref[0:1, :]

    return pl.pallas_call(
        body,
        grid=(nblk,),
        in_specs=[
            pl.BlockSpec((BLK, D), lambda i: (i, 0)),
            pl.BlockSpec((BLK, D), lambda i: (i, 0)),
            pl.BlockSpec((BLK, D), lambda i: (i, 0)),
            pl.BlockSpec((BLK, D), lambda i: (i, 0)),
            pl.BlockSpec((BLK, D), lambda i: (i, 0)),
            pl.BlockSpec((1, 1, BLK), lambda i: (i, 0, 0)),
            pl.BlockSpec((D, D), lambda i: (0, 0)),
            pl.BlockSpec((1, D), lambda i: (0, 0)),
            pl.BlockSpec((D, GP), lambda i: (0, 0)),
            pl.BlockSpec((1, GP), lambda i: (0, 0)),
        ],
        out_specs=pl.BlockSpec((GP, D), lambda i: (0, 0)),
        out_shape=jax.ShapeDtypeStruct((GP, D), jnp.float32),
        scratch_shapes=[
            pltpu.VMEM((GP, D), jnp.float32),
            pltpu.VMEM((GP, D), jnp.float32),
        ],
    )(pa, pb, s_prev, dega, degb, batch3d, Wh1, bh1, Wh2p, bh2p)


def kernel(x, edge_index, batch, W0, b0, W1, b1, W2, b2, Wh1, bh1, Wh2, bh2):
    N, D = x.shape
    E = edge_index.shape[1]
    G = 64
    EP = ((E + GRP - 1) // GRP) * GRP
    pad = EP - E
    row = jnp.concatenate(
        [edge_index[0], jnp.zeros((pad,), jnp.int32)]).reshape(EP // CHUNK, CHUNK)
    col = jnp.concatenate(
        [edge_index[1], jnp.full((pad,), N, jnp.int32)]).reshape(EP // CHUNK, CHUNK)

    zeros_d = jnp.zeros((CHUNK, D), jnp.float32)
    ones_d = jnp.ones((CHUNK, D), jnp.float32)
    batch3d = batch.reshape(N // 1000, 1, 1000)
    b0r = b0.reshape(1, D)
    b1r = b1.reshape(1, D)
    b2r = b2.reshape(1, D)
    bh1r = bh1.reshape(1, D)
    Wh2p = jnp.pad(Wh2, ((0, 0), (0, 128 - Wh2.shape[1])))
    bh2p = jnp.broadcast_to(bh2.reshape(1, 1), (1, 128))

    deg = _make_deg(N, EP, D)
    gath = _make_gather(N, EP, D)
    scat = _make_scatter(N, EP, D)

    dega, degb = deg(col, ones_d, zeros_d)

    s0 = _mm_first(x, W0, b0r, dega, degb)
    p0a, p0b = scat(gath(s0, row), col, zeros_d)
    s1 = _mm_mid(p0a, p0b, s0, W1, b1r, dega, degb)
    p1a, p1b = scat(gath(s1, row), col, zeros_d)
    s2 = _mm_mid(p1a, p1b, s1, W2, b2r, dega, degb)
    p2a, p2b = scat(gath(s2, row), col, zeros_d)

    out = _final(p2a, p2b, s2, dega, degb, batch3d, Wh1, bh1r, Wh2p, bh2p)
    return out[:G, 0]
